# Initial kernel scaffold; baseline (speedup 1.0000x reference)
import jax
import jax.numpy as jnp
from jax.experimental import pallas as pl


def kernel(x, edge_index, W_in1, b_in1, W_in2, b_in2, l0_W1, l0_b1, l0_W2, l0_b2, l1_W1, l1_b1, l1_W2, l1_b2, Kmat, out_W1, out_b1, out_W2, out_b2, bI, bJ, bK, bC):
    raise NotImplementedError("write your pallas kernel here")


# speedup vs baseline: 1.9531x; 1.9531x over previous
import functools
import jax
import jax.numpy as jnp
import numpy as np
from jax import lax
from jax.experimental import pallas as pl
from jax.experimental.pallas import tpu as pltpu

N_NODES = 10000
N_EDGES = 320000
IN_DIM = 128
HIDDEN = 128
OUT_DIM = 32
D_A = 14
DP = 16
NNZ = 64


def _sig(x):
    return 1.0 / (1.0 + jnp.exp(-x))


def _in_mlp_body(x_ref, w1_ref, b1_ref, w2_ref, b2_ref, o_ref):
    h1 = jnp.dot(x_ref[...], w1_ref[...], preferred_element_type=jnp.float32)
    h1 = h1 + b1_ref[...]
    h1 = h1 * _sig(h1)
    h2 = jnp.dot(h1, w2_ref[...], preferred_element_type=jnp.float32)
    o_ref[...] = h2 + b2_ref[...]


def _in_mlp(x, W1, b1, W2p, b2p):
    B = 2000
    grid = (N_NODES // B,)
    return pl.pallas_call(
        _in_mlp_body,
        grid=grid,
        in_specs=[
            pl.BlockSpec((B, IN_DIM), lambda i: (i, 0)),
            pl.BlockSpec((IN_DIM, HIDDEN), lambda i: (0, 0)),
            pl.BlockSpec((1, HIDDEN), lambda i: (0, 0)),
            pl.BlockSpec((HIDDEN, DP), lambda i: (0, 0)),
            pl.BlockSpec((1, DP), lambda i: (0, 0)),
        ],
        out_specs=pl.BlockSpec((B, DP), lambda i: (i, 0)),
        out_shape=jax.ShapeDtypeStruct((N_NODES, DP), jnp.float32),
    )(x, W1, b1, W2p, b2p)


def _edge_body(g_ref, w1_ref, b1_ref, si_ref, sj_ref, skc_ref, wc_ref,
               w2_ref, b2_ref, o_ref):
    g = g_ref[...]
    hs = g[:, :DP]
    ht = g[:, DP:]
    hid = jnp.dot(g, w1_ref[...], preferred_element_type=jnp.float32)
    t1 = jnp.dot(hs, si_ref[...], preferred_element_type=jnp.float32)
    t2 = jnp.dot(ht, sj_ref[...], preferred_element_type=jnp.float32)
    br = jnp.dot(t1 * t2, skc_ref[...], preferred_element_type=jnp.float32)
    hid = hid + jnp.dot(br, wc_ref[...], preferred_element_type=jnp.float32)
    hid = hid + b1_ref[...]
    hid = hid * _sig(hid)
    m = jnp.dot(hid, w2_ref[...], preferred_element_type=jnp.float32)
    o_ref[...] = m + b2_ref[...]


def _edge_mlp(g, W1eff, b1, SI, SJ, SKC, W1c, W2p, b2p):
    B = 2000
    grid = (N_EDGES // B,)
    return pl.pallas_call(
        _edge_body,
        grid=grid,
        in_specs=[
            pl.BlockSpec((B, 2 * DP), lambda i: (i, 0)),
            pl.BlockSpec((2 * DP, HIDDEN), lambda i: (0, 0)),
            pl.BlockSpec((1, HIDDEN), lambda i: (0, 0)),
            pl.BlockSpec((DP, NNZ), lambda i: (0, 0)),
            pl.BlockSpec((DP, NNZ), lambda i: (0, 0)),
            pl.BlockSpec((NNZ, DP), lambda i: (0, 0)),
            pl.BlockSpec((DP, HIDDEN), lambda i: (0, 0)),
            pl.BlockSpec((HIDDEN, DP), lambda i: (0, 0)),
            pl.BlockSpec((1, DP), lambda i: (0, 0)),
        ],
        out_specs=pl.BlockSpec((B, DP), lambda i: (i, 0)),
        out_shape=jax.ShapeDtypeStruct((N_EDGES, DP), jnp.float32),
    )(g, W1eff, b1, SI, SJ, SKC, W1c, W2p, b2p)


def _pool_body(h_ref, k_ref, w1_ref, b1_ref, w2_ref, b2_ref, o_ref):
    h = h_ref[...]
    hk = jnp.dot(h, k_ref[...], preferred_element_type=jnp.float32)
    killing = jnp.sum(hk * h, axis=1, keepdims=True)
    ph = jnp.sum(h, axis=0, keepdims=True) * (1.0 / N_NODES)
    pk = jnp.sum(killing, axis=0, keepdims=True) * (1.0 / N_NODES)
    p = jnp.concatenate([ph, pk, jnp.zeros((1, 2 * DP - DP - 1), jnp.float32)],
                        axis=1)
    z = jnp.dot(p, w1_ref[...], preferred_element_type=jnp.float32)
    z = z + b1_ref[...]
    z = z * _sig(z)
    o_ref[...] = jnp.dot(z, w2_ref[...],
                         preferred_element_type=jnp.float32) + b2_ref[...]


def _pool(h, Kp, Wo1p, bo1, Wo2, bo2):
    return pl.pallas_call(
        _pool_body,
        in_specs=[
            pl.BlockSpec((N_NODES, DP), lambda: (0, 0)),
            pl.BlockSpec((DP, DP), lambda: (0, 0)),
            pl.BlockSpec((2 * DP, HIDDEN), lambda: (0, 0)),
            pl.BlockSpec((1, HIDDEN), lambda: (0, 0)),
            pl.BlockSpec((HIDDEN, OUT_DIM), lambda: (0, 0)),
            pl.BlockSpec((1, OUT_DIM), lambda: (0, 0)),
        ],
        out_specs=pl.BlockSpec((1, OUT_DIM), lambda: (0, 0)),
        out_shape=jax.ShapeDtypeStruct((1, OUT_DIM), jnp.float32),
    )(h, Kp, Wo1p, bo1, Wo2, bo2)


def kernel(x, edge_index, W_in1, b_in1, W_in2, b_in2, l0_W1, l0_b1, l0_W2,
           l0_b2, l1_W1, l1_b1, l1_W2, l1_b2, Kmat, out_W1, out_b1, out_W2,
           out_b2, bI, bJ, bK, bC):
    src = edge_index[0]
    tgt = edge_index[1]

    pad = lambda a, r: jnp.zeros((r, a.shape[1]), a.dtype).at[:a.shape[0]].set(a)
    padc = lambda a, c: jnp.zeros((a.shape[0], c), a.dtype).at[:, :a.shape[1]].set(a)

    W_in2p = padc(W_in2, DP)
    b_in2p = padc(b_in2[None, :], DP)

    SI = jnp.zeros((DP, NNZ), jnp.float32).at[bI, jnp.arange(NNZ)].set(1.0)
    SJ = jnp.zeros((DP, NNZ), jnp.float32).at[bJ, jnp.arange(NNZ)].set(1.0)
    SKC = jnp.zeros((NNZ, DP), jnp.float32).at[jnp.arange(NNZ), bK].set(bC)

    layers = []
    for (W1, b1, W2, b2) in ((l0_W1, l0_b1, l0_W2, l0_b2),
                             (l1_W1, l1_b1, l1_W2, l1_b2)):
        W1eff = jnp.concatenate([pad(W1[:D_A], DP), pad(W1[D_A:2 * D_A], DP)],
                                axis=0)
        W1c = pad(W1[2 * D_A:], DP)
        W2p = padc(W2, DP)
        b2p = padc(b2[None, :], DP)
        layers.append((W1eff, b1[None, :], W1c, W2p, b2p))

    Kp = pad(padc(Kmat, DP), DP)
    Wo1p = pad(out_W1, 2 * DP).at[D_A + 1:].set(0.0)
    Wo1p = jnp.zeros((2 * DP, HIDDEN), jnp.float32)
    Wo1p = Wo1p.at[:D_A].set(out_W1[:D_A])
    Wo1p = Wo1p.at[DP].set(out_W1[D_A])

    h = _in_mlp(x, W_in1, b_in1[None, :], W_in2p, b_in2p)

    for (W1eff, b1r, W1c, W2p, b2p) in layers:
        hs = jnp.take(h, src, axis=0)
        ht = jnp.take(h, tgt, axis=0)
        g = jnp.concatenate([hs, ht], axis=1)
        m = _edge_mlp(g, W1eff, b1r, SI, SJ, SKC, W1c, W2p, b2p)
        h = h + jax.ops.segment_sum(m, tgt, num_segments=N_NODES)

    return _pool(h, Kp, Wo1p, out_b1[None, :], out_W2, out_b2[None, :])


# speedup vs baseline: 4.0380x; 2.0675x over previous
import functools
import jax
import jax.numpy as jnp
import numpy as np
from jax import lax
from jax.experimental import pallas as pl
from jax.experimental.pallas import tpu as pltpu
from jax.experimental.pallas import tpu_sc as plsc

N_NODES = 10000
N_EDGES = 320000
IN_DIM = 128
HIDDEN = 128
OUT_DIM = 32
D_A = 14
DP = 16
NNZ = 64

NC = 2
NS = 16
NW = NC * NS
EW = N_EDGES // NW
C = 80
CHUNKS = EW // C
GC = 25
GE = GC * C
NG = CHUNKS // GC


def _gather_body(h_hbm, src_hbm, tgt_hbm, gs_hbm, gt_hbm,
                 idxbuf, rows, gsem, osem0, osem1):
    c = lax.axis_index("c")
    s = lax.axis_index("s")
    wid = s * NC + c
    ebase = wid * EW
    for idx_hbm, out_hbm in ((src_hbm, gs_hbm), (tgt_hbm, gt_hbm)):
        pltpu.sync_copy(idx_hbm.at[wid], idxbuf)
        osems = (osem0, osem1)
        descs = [None] * NG
        for g in range(NG):
            b = g & 1
            if g >= 2:
                descs[g - 2].wait()

            def fire(j, carry, g=g, b=b):
                pltpu.async_copy(h_hbm.at[idxbuf.at[g * GC + j]],
                                 rows.at[b, pl.ds(j * C, C)], gsem)
                return carry

            lax.fori_loop(0, GC, fire, 0)
            pltpu.make_async_copy(out_hbm.at[pl.ds(ebase + g * GE, GE)],
                                  rows.at[b], gsem).wait()
            descs[g] = pltpu.async_copy(
                rows.at[b], out_hbm.at[pl.ds(ebase + g * GE, GE)], osems[b])
        descs[NG - 2].wait()
        descs[NG - 1].wait()


def _sc_gather(h, src2d, tgt2d):
    mesh = plsc.VectorSubcoreMesh(core_axis_name="c", subcore_axis_name="s",
                                  num_cores=NC, num_subcores=NS)
    f = pl.kernel(
        _gather_body,
        out_type=(jax.ShapeDtypeStruct((N_EDGES, DP), jnp.float32),
                  jax.ShapeDtypeStruct((N_EDGES, DP), jnp.float32)),
        mesh=mesh,
        scratch_types=[
            pltpu.VMEM((CHUNKS, C), jnp.int32),
            pltpu.VMEM((2, GE, DP), jnp.float32),
            pltpu.SemaphoreType.DMA,
            pltpu.SemaphoreType.DMA,
            pltpu.SemaphoreType.DMA,
        ],
        compiler_params=pltpu.CompilerParams(use_tc_tiling_on_sc=False),
    )
    return f(h, src2d, tgt2d)


def _sig(x):
    return 1.0 / (1.0 + jnp.exp(-x))


def _in_mlp_body(x_ref, w1_ref, b1_ref, w2_ref, b2_ref, o_ref):
    h1 = jnp.dot(x_ref[...], w1_ref[...], preferred_element_type=jnp.float32)
    h1 = h1 + b1_ref[...]
    h1 = h1 * _sig(h1)
    h2 = jnp.dot(h1, w2_ref[...], preferred_element_type=jnp.float32)
    o_ref[...] = h2 + b2_ref[...]


def _in_mlp(x, W1, b1, W2p, b2p):
    B = 2000
    grid = (N_NODES // B,)
    return pl.pallas_call(
        _in_mlp_body,
        grid=grid,
        in_specs=[
            pl.BlockSpec((B, IN_DIM), lambda i: (i, 0)),
            pl.BlockSpec((IN_DIM, HIDDEN), lambda i: (0, 0)),
            pl.BlockSpec((1, HIDDEN), lambda i: (0, 0)),
            pl.BlockSpec((HIDDEN, DP), lambda i: (0, 0)),
            pl.BlockSpec((1, DP), lambda i: (0, 0)),
        ],
        out_specs=pl.BlockSpec((B, DP), lambda i: (i, 0)),
        out_shape=jax.ShapeDtypeStruct((N_NODES, DP), jnp.float32),
    )(x, W1, b1, W2p, b2p)


def _edge_body(gs_ref, gt_ref, w1_ref, b1_ref, si_ref, sj_ref, skc_ref,
               wc_ref, w2_ref, b2_ref, o_ref):
    hs = gs_ref[...]
    ht = gt_ref[...]
    hid = jnp.dot(hs, w1_ref[:DP], preferred_element_type=jnp.float32)
    hid = hid + jnp.dot(ht, w1_ref[DP:], preferred_element_type=jnp.float32)
    t1 = jnp.dot(hs, si_ref[...], preferred_element_type=jnp.float32)
    t2 = jnp.dot(ht, sj_ref[...], preferred_element_type=jnp.float32)
    br = jnp.dot(t1 * t2, skc_ref[...], preferred_element_type=jnp.float32)
    hid = hid + jnp.dot(br, wc_ref[...], preferred_element_type=jnp.float32)
    hid = hid + b1_ref[...]
    hid = hid * _sig(hid)
    m = jnp.dot(hid, w2_ref[...], preferred_element_type=jnp.float32)
    o_ref[...] = m + b2_ref[...]


def _edge_mlp(gs, gt, W1eff, b1, SI, SJ, SKC, W1c, W2p, b2p):
    B = 2000
    grid = (N_EDGES // B,)
    return pl.pallas_call(
        _edge_body,
        grid=grid,
        in_specs=[
            pl.BlockSpec((B, DP), lambda i: (i, 0)),
            pl.BlockSpec((B, DP), lambda i: (i, 0)),
            pl.BlockSpec((2 * DP, HIDDEN), lambda i: (0, 0)),
            pl.BlockSpec((1, HIDDEN), lambda i: (0, 0)),
            pl.BlockSpec((DP, NNZ), lambda i: (0, 0)),
            pl.BlockSpec((DP, NNZ), lambda i: (0, 0)),
            pl.BlockSpec((NNZ, DP), lambda i: (0, 0)),
            pl.BlockSpec((DP, HIDDEN), lambda i: (0, 0)),
            pl.BlockSpec((HIDDEN, DP), lambda i: (0, 0)),
            pl.BlockSpec((1, DP), lambda i: (0, 0)),
        ],
        out_specs=pl.BlockSpec((B, DP), lambda i: (i, 0)),
        out_shape=jax.ShapeDtypeStruct((N_EDGES, DP), jnp.float32),
    )(gs, gt, W1eff, b1, SI, SJ, SKC, W1c, W2p, b2p)


def _pool_body(h_ref, k_ref, w1_ref, b1_ref, w2_ref, b2_ref, o_ref):
    h = h_ref[...]
    hk = jnp.dot(h, k_ref[...], preferred_element_type=jnp.float32)
    killing = jnp.sum(hk * h, axis=1, keepdims=True)
    ph = jnp.sum(h, axis=0, keepdims=True) * (1.0 / N_NODES)
    pk = jnp.sum(killing, axis=0, keepdims=True) * (1.0 / N_NODES)
    p = jnp.concatenate([ph, pk, jnp.zeros((1, 2 * DP - DP - 1), jnp.float32)],
                        axis=1)
    z = jnp.dot(p, w1_ref[...], preferred_element_type=jnp.float32)
    z = z + b1_ref[...]
    z = z * _sig(z)
    o_ref[...] = jnp.dot(z, w2_ref[...],
                         preferred_element_type=jnp.float32) + b2_ref[...]


def _pool(h, Kp, Wo1p, bo1, Wo2, bo2):
    return pl.pallas_call(
        _pool_body,
        in_specs=[
            pl.BlockSpec((N_NODES, DP), lambda: (0, 0)),
            pl.BlockSpec((DP, DP), lambda: (0, 0)),
            pl.BlockSpec((2 * DP, HIDDEN), lambda: (0, 0)),
            pl.BlockSpec((1, HIDDEN), lambda: (0, 0)),
            pl.BlockSpec((HIDDEN, OUT_DIM), lambda: (0, 0)),
            pl.BlockSpec((1, OUT_DIM), lambda: (0, 0)),
        ],
        out_specs=pl.BlockSpec((1, OUT_DIM), lambda: (0, 0)),
        out_shape=jax.ShapeDtypeStruct((1, OUT_DIM), jnp.float32),
    )(h, Kp, Wo1p, bo1, Wo2, bo2)


def kernel(x, edge_index, W_in1, b_in1, W_in2, b_in2, l0_W1, l0_b1, l0_W2,
           l0_b2, l1_W1, l1_b1, l1_W2, l1_b2, Kmat, out_W1, out_b1, out_W2,
           out_b2, bI, bJ, bK, bC):
    src = edge_index[0]
    tgt = edge_index[1]

    pad = lambda a, r: jnp.zeros((r, a.shape[1]), a.dtype).at[:a.shape[0]].set(a)
    padc = lambda a, c: jnp.zeros((a.shape[0], c), a.dtype).at[:, :a.shape[1]].set(a)

    W_in2p = padc(W_in2, DP)
    b_in2p = padc(b_in2[None, :], DP)

    SI = jnp.zeros((DP, NNZ), jnp.float32).at[bI, jnp.arange(NNZ)].set(1.0)
    SJ = jnp.zeros((DP, NNZ), jnp.float32).at[bJ, jnp.arange(NNZ)].set(1.0)
    SKC = jnp.zeros((NNZ, DP), jnp.float32).at[jnp.arange(NNZ), bK].set(bC)

    layers = []
    for (W1, b1, W2, b2) in ((l0_W1, l0_b1, l0_W2, l0_b2),
                             (l1_W1, l1_b1, l1_W2, l1_b2)):
        W1eff = jnp.concatenate([pad(W1[:D_A], DP), pad(W1[D_A:2 * D_A], DP)],
                                axis=0)
        W1c = pad(W1[2 * D_A:], DP)
        W2p = padc(W2, DP)
        b2p = padc(b2[None, :], DP)
        layers.append((W1eff, b1[None, :], W1c, W2p, b2p))

    Kp = pad(padc(Kmat, DP), DP)
    Wo1p = pad(out_W1, 2 * DP).at[D_A + 1:].set(0.0)
    Wo1p = jnp.zeros((2 * DP, HIDDEN), jnp.float32)
    Wo1p = Wo1p.at[:D_A].set(out_W1[:D_A])
    Wo1p = Wo1p.at[DP].set(out_W1[D_A])

    src2d = src.reshape(NW, CHUNKS, C)
    tgt2d = tgt.reshape(NW, CHUNKS, C)
    h = _in_mlp(x, W_in1, b_in1[None, :], W_in2p, b_in2p)

    for (W1eff, b1r, W1c, W2p, b2p) in layers:
        gs, gt = _sc_gather(h, src2d, tgt2d)
        m = _edge_mlp(gs, gt, W1eff, b1r, SI, SJ, SKC, W1c, W2p, b2p)
        h = h + jax.ops.segment_sum(m, tgt, num_segments=N_NODES)

    return _pool(h, Kp, Wo1p, out_b1[None, :], out_W2, out_b2[None, :])


# speedup vs baseline: 7.8702x; 1.9490x over previous
import functools
import jax
import jax.numpy as jnp
import numpy as np
from jax import lax
from jax.experimental import pallas as pl
from jax.experimental.pallas import tpu as pltpu
from jax.experimental.pallas import tpu_sc as plsc

N_NODES = 10000
N_EDGES = 320000
IN_DIM = 128
HIDDEN = 128
OUT_DIM = 32
D_A = 14
DP = 16
NNZ = 64

NC = 2
NS = 16
NW = NC * NS
EW = N_EDGES // NW
C = 80
CHUNKS = EW // C
GC = 25
GE = GC * C
NG = CHUNKS // GC


def _gather_body(h_hbm, src_hbm, tgt_hbm, gs_hbm, gt_hbm,
                 idxbuf, rows, gsem, osem0, osem1):
    c = lax.axis_index("c")
    s = lax.axis_index("s")
    wid = s * NC + c
    ebase = wid * EW
    for idx_hbm, out_hbm in ((src_hbm, gs_hbm), (tgt_hbm, gt_hbm)):
        pltpu.sync_copy(idx_hbm.at[wid], idxbuf)
        osems = (osem0, osem1)
        descs = [None] * NG
        for g in range(NG):
            b = g & 1
            if g >= 2:
                descs[g - 2].wait()

            def fire(j, carry, g=g, b=b):
                pltpu.async_copy(h_hbm.at[idxbuf.at[g * GC + j]],
                                 rows.at[b, pl.ds(j * C, C)], gsem)
                return carry

            lax.fori_loop(0, GC, fire, 0)
            pltpu.make_async_copy(out_hbm.at[pl.ds(ebase + g * GE, GE)],
                                  rows.at[b], gsem).wait()
            descs[g] = pltpu.async_copy(
                rows.at[b], out_hbm.at[pl.ds(ebase + g * GE, GE)], osems[b])
        descs[NG - 2].wait()
        descs[NG - 1].wait()


def _sc_gather(h, src2d, tgt2d):
    mesh = plsc.VectorSubcoreMesh(core_axis_name="c", subcore_axis_name="s",
                                  num_cores=NC, num_subcores=NS)
    f = pl.kernel(
        _gather_body,
        out_type=(jax.ShapeDtypeStruct((N_EDGES, DP), jnp.float32),
                  jax.ShapeDtypeStruct((N_EDGES, DP), jnp.float32)),
        mesh=mesh,
        scratch_types=[
            pltpu.VMEM((CHUNKS, C), jnp.int32),
            pltpu.VMEM((2, GE, DP), jnp.float32),
            pltpu.SemaphoreType.DMA,
            pltpu.SemaphoreType.DMA,
            pltpu.SemaphoreType.DMA,
        ],
        compiler_params=pltpu.CompilerParams(use_tc_tiling_on_sc=False),
    )
    return f(h, src2d, tgt2d)


def _sig(x):
    return 1.0 / (1.0 + jnp.exp(-x))


def _in_mlp_body(x_ref, w1_ref, b1_ref, w2_ref, b2_ref, o_ref):
    h1 = jnp.dot(x_ref[...], w1_ref[...], preferred_element_type=jnp.float32)
    h1 = h1 + b1_ref[...]
    h1 = h1 * _sig(h1)
    h2 = jnp.dot(h1, w2_ref[...], preferred_element_type=jnp.float32)
    o_ref[...] = h2 + b2_ref[...]


def _in_mlp(x, W1, b1, W2p, b2p):
    B = 2000
    grid = (N_NODES // B,)
    return pl.pallas_call(
        _in_mlp_body,
        grid=grid,
        in_specs=[
            pl.BlockSpec((B, IN_DIM), lambda i: (i, 0)),
            pl.BlockSpec((IN_DIM, HIDDEN), lambda i: (0, 0)),
            pl.BlockSpec((1, HIDDEN), lambda i: (0, 0)),
            pl.BlockSpec((HIDDEN, DP), lambda i: (0, 0)),
            pl.BlockSpec((1, DP), lambda i: (0, 0)),
        ],
        out_specs=pl.BlockSpec((B, DP), lambda i: (i, 0)),
        out_shape=jax.ShapeDtypeStruct((N_NODES, DP), jnp.float32),
    )(x, W1, b1, W2p, b2p)


NR = N_NODES // NS


def _scatter_body(h_hbm, m_hbm, tgt_hbm, parts_hbm,
                  idxbuf, mbuf, zbuf, acc, msem, ssem):
    c = lax.axis_index("c")
    s = lax.axis_index("s")
    wid = s * NC + c
    ebase = wid * EW

    def zloop(j, carry):
        zbuf[j, :] = jnp.zeros((DP,), jnp.float32)
        return carry

    lax.fori_loop(0, NR, zloop, 0)

    @pl.when(c == 0)
    def _():
        pltpu.sync_copy(h_hbm.at[pl.ds(s * NR, NR)], zbuf)

    pltpu.sync_copy(zbuf, acc.at[pl.ds(s * NR, NR)])
    pltpu.sync_copy(tgt_hbm.at[wid], idxbuf)
    plsc.subcore_barrier()

    ld = pltpu.async_copy(m_hbm.at[pl.ds(ebase, GE)], mbuf.at[0], msem)
    for g in range(NG):
        b = g & 1
        ld.wait()
        if g + 1 < NG:
            ld = pltpu.async_copy(
                m_hbm.at[pl.ds(ebase + (g + 1) * GE, GE)], mbuf.at[b ^ 1],
                msem)

        def fire(j, carry, g=g, b=b):
            pltpu.async_copy(mbuf.at[b, pl.ds(j * C, C)],
                             acc.at[idxbuf.at[g * GC + j]], ssem, add=True)
            return carry

        lax.fori_loop(0, GC, fire, 0)
        pltpu.make_async_copy(mbuf.at[b], acc.at[pl.ds(0, GE)], ssem).wait()

    plsc.subcore_barrier()
    pltpu.sync_copy(acc.at[pl.ds(s * NR, NR)],
                    parts_hbm.at[c, pl.ds(s * NR, NR)])


def _sc_scatter(h, m, tgt3d):
    mesh = plsc.VectorSubcoreMesh(core_axis_name="c", subcore_axis_name="s",
                                  num_cores=NC, num_subcores=NS)
    f = pl.kernel(
        _scatter_body,
        out_type=jax.ShapeDtypeStruct((NC, N_NODES, DP), jnp.float32),
        mesh=mesh,
        scratch_types=[
            pltpu.VMEM((CHUNKS, C), jnp.int32),
            pltpu.VMEM((2, GE, DP), jnp.float32),
            pltpu.VMEM((NR, DP), jnp.float32),
            pltpu.VMEM_SHARED((N_NODES, DP), jnp.float32),
            pltpu.SemaphoreType.DMA,
            pltpu.SemaphoreType.DMA,
        ],
        compiler_params=pltpu.CompilerParams(use_tc_tiling_on_sc=False),
    )
    return f(h, m, tgt3d)


def _add_body(a_ref, b_ref, o_ref):
    o_ref[...] = a_ref[0] + b_ref[0]


def _combine(parts):
    B = 2000
    return pl.pallas_call(
        _add_body,
        grid=(N_NODES // B,),
        in_specs=[
            pl.BlockSpec((1, B, DP), lambda i: (0, i, 0)),
            pl.BlockSpec((1, B, DP), lambda i: (1, i, 0)),
        ],
        out_specs=pl.BlockSpec((B, DP), lambda i: (i, 0)),
        out_shape=jax.ShapeDtypeStruct((N_NODES, DP), jnp.float32),
    )(parts, parts)


def _edge_body(gs_ref, gt_ref, w1_ref, b1_ref, si_ref, sj_ref, skc_ref,
               wc_ref, w2_ref, b2_ref, o_ref):
    hs = gs_ref[...]
    ht = gt_ref[...]
    hid = jnp.dot(hs, w1_ref[:DP], preferred_element_type=jnp.float32)
    hid = hid + jnp.dot(ht, w1_ref[DP:], preferred_element_type=jnp.float32)
    t1 = jnp.dot(hs, si_ref[...], preferred_element_type=jnp.float32)
    t2 = jnp.dot(ht, sj_ref[...], preferred_element_type=jnp.float32)
    br = jnp.dot(t1 * t2, skc_ref[...], preferred_element_type=jnp.float32)
    hid = hid + jnp.dot(br, wc_ref[...], preferred_element_type=jnp.float32)
    hid = hid + b1_ref[...]
    hid = hid * _sig(hid)
    m = jnp.dot(hid, w2_ref[...], preferred_element_type=jnp.float32)
    o_ref[...] = m + b2_ref[...]


def _edge_mlp(gs, gt, W1eff, b1, SI, SJ, SKC, W1c, W2p, b2p):
    B = 2000
    grid = (N_EDGES // B,)
    return pl.pallas_call(
        _edge_body,
        grid=grid,
        in_specs=[
            pl.BlockSpec((B, DP), lambda i: (i, 0)),
            pl.BlockSpec((B, DP), lambda i: (i, 0)),
            pl.BlockSpec((2 * DP, HIDDEN), lambda i: (0, 0)),
            pl.BlockSpec((1, HIDDEN), lambda i: (0, 0)),
            pl.BlockSpec((DP, NNZ), lambda i: (0, 0)),
            pl.BlockSpec((DP, NNZ), lambda i: (0, 0)),
            pl.BlockSpec((NNZ, DP), lambda i: (0, 0)),
            pl.BlockSpec((DP, HIDDEN), lambda i: (0, 0)),
            pl.BlockSpec((HIDDEN, DP), lambda i: (0, 0)),
            pl.BlockSpec((1, DP), lambda i: (0, 0)),
        ],
        out_specs=pl.BlockSpec((B, DP), lambda i: (i, 0)),
        out_shape=jax.ShapeDtypeStruct((N_EDGES, DP), jnp.float32),
    )(gs, gt, W1eff, b1, SI, SJ, SKC, W1c, W2p, b2p)


def _pool_body(h_ref, k_ref, w1_ref, b1_ref, w2_ref, b2_ref, o_ref):
    h = h_ref[...]
    hk = jnp.dot(h, k_ref[...], preferred_element_type=jnp.float32)
    killing = jnp.sum(hk * h, axis=1, keepdims=True)
    ph = jnp.sum(h, axis=0, keepdims=True) * (1.0 / N_NODES)
    pk = jnp.sum(killing, axis=0, keepdims=True) * (1.0 / N_NODES)
    p = jnp.concatenate([ph, pk, jnp.zeros((1, 2 * DP - DP - 1), jnp.float32)],
                        axis=1)
    z = jnp.dot(p, w1_ref[...], preferred_element_type=jnp.float32)
    z = z + b1_ref[...]
    z = z * _sig(z)
    o_ref[...] = jnp.dot(z, w2_ref[...],
                         preferred_element_type=jnp.float32) + b2_ref[...]


def _pool(h, Kp, Wo1p, bo1, Wo2, bo2):
    return pl.pallas_call(
        _pool_body,
        in_specs=[
            pl.BlockSpec((N_NODES, DP), lambda: (0, 0)),
            pl.BlockSpec((DP, DP), lambda: (0, 0)),
            pl.BlockSpec((2 * DP, HIDDEN), lambda: (0, 0)),
            pl.BlockSpec((1, HIDDEN), lambda: (0, 0)),
            pl.BlockSpec((HIDDEN, OUT_DIM), lambda: (0, 0)),
            pl.BlockSpec((1, OUT_DIM), lambda: (0, 0)),
        ],
        out_specs=pl.BlockSpec((1, OUT_DIM), lambda: (0, 0)),
        out_shape=jax.ShapeDtypeStruct((1, OUT_DIM), jnp.float32),
    )(h, Kp, Wo1p, bo1, Wo2, bo2)


def kernel(x, edge_index, W_in1, b_in1, W_in2, b_in2, l0_W1, l0_b1, l0_W2,
           l0_b2, l1_W1, l1_b1, l1_W2, l1_b2, Kmat, out_W1, out_b1, out_W2,
           out_b2, bI, bJ, bK, bC):
    src = edge_index[0]
    tgt = edge_index[1]

    pad = lambda a, r: jnp.zeros((r, a.shape[1]), a.dtype).at[:a.shape[0]].set(a)
    padc = lambda a, c: jnp.zeros((a.shape[0], c), a.dtype).at[:, :a.shape[1]].set(a)

    W_in2p = padc(W_in2, DP)
    b_in2p = padc(b_in2[None, :], DP)

    SI = jnp.zeros((DP, NNZ), jnp.float32).at[bI, jnp.arange(NNZ)].set(1.0)
    SJ = jnp.zeros((DP, NNZ), jnp.float32).at[bJ, jnp.arange(NNZ)].set(1.0)
    SKC = jnp.zeros((NNZ, DP), jnp.float32).at[jnp.arange(NNZ), bK].set(bC)

    layers = []
    for (W1, b1, W2, b2) in ((l0_W1, l0_b1, l0_W2, l0_b2),
                             (l1_W1, l1_b1, l1_W2, l1_b2)):
        W1eff = jnp.concatenate([pad(W1[:D_A], DP), pad(W1[D_A:2 * D_A], DP)],
                                axis=0)
        W1c = pad(W1[2 * D_A:], DP)
        W2p = padc(W2, DP)
        b2p = padc(b2[None, :], DP)
        layers.append((W1eff, b1[None, :], W1c, W2p, b2p))

    Kp = pad(padc(Kmat, DP), DP)
    Wo1p = pad(out_W1, 2 * DP).at[D_A + 1:].set(0.0)
    Wo1p = jnp.zeros((2 * DP, HIDDEN), jnp.float32)
    Wo1p = Wo1p.at[:D_A].set(out_W1[:D_A])
    Wo1p = Wo1p.at[DP].set(out_W1[D_A])

    src2d = src.reshape(NW, CHUNKS, C)
    tgt2d = tgt.reshape(NW, CHUNKS, C)
    h = _in_mlp(x, W_in1, b_in1[None, :], W_in2p, b_in2p)

    for (W1eff, b1r, W1c, W2p, b2p) in layers:
        gs, gt = _sc_gather(h, src2d, tgt2d)
        m = _edge_mlp(gs, gt, W1eff, b1r, SI, SJ, SKC, W1c, W2p, b2p)
        parts = _sc_scatter(h, m, tgt2d)
        h = _combine(parts)

    return _pool(h, Kp, Wo1p, out_b1[None, :], out_W2, out_b2[None, :])


# speedup vs baseline: 16.4484x; 2.0900x over previous
import functools
import jax
import jax.numpy as jnp
import numpy as np
from jax import lax
from jax.experimental import pallas as pl
from jax.experimental.pallas import tpu as pltpu
from jax.experimental.pallas import tpu_sc as plsc

N_NODES = 10000
N_EDGES = 320000
IN_DIM = 128
HIDDEN = 128
OUT_DIM = 32
D_A = 14
DP = 16
NNZ = 64

NC = 2
NS = 16
NW = NC * NS
EW = N_EDGES // NW
C = 80
CHUNKS = EW // C
GC = 25
GE = GC * C
NG = CHUNKS // GC


def _gather_body(h_hbm, src_hbm, tgt_hbm, gs_hbm, gt_hbm,
                 idxbuf, rows, gsem, osem0, osem1):
    c = lax.axis_index("c")
    s = lax.axis_index("s")
    wid = s * NC + c
    ebase = wid * EW
    for idx_hbm, out_hbm in ((src_hbm, gs_hbm), (tgt_hbm, gt_hbm)):
        pltpu.sync_copy(idx_hbm.at[wid], idxbuf)
        osems = (osem0, osem1)
        descs = [None] * NG
        for g in range(NG):
            b = g & 1
            if g >= 2:
                descs[g - 2].wait()

            def fire(j, carry, g=g, b=b):
                pltpu.async_copy(h_hbm.at[idxbuf.at[g * GC + j]],
                                 rows.at[b, pl.ds(j * C, C)], gsem)
                return carry

            lax.fori_loop(0, GC, fire, 0)
            pltpu.make_async_copy(out_hbm.at[pl.ds(ebase + g * GE, GE)],
                                  rows.at[b], gsem).wait()
            descs[g] = pltpu.async_copy(
                rows.at[b], out_hbm.at[pl.ds(ebase + g * GE, GE)], osems[b])
        descs[NG - 2].wait()
        descs[NG - 1].wait()


def _sc_gather(h, src2d, tgt2d):
    mesh = plsc.VectorSubcoreMesh(core_axis_name="c", subcore_axis_name="s",
                                  num_cores=NC, num_subcores=NS)
    f = pl.kernel(
        _gather_body,
        out_type=(jax.ShapeDtypeStruct((N_EDGES, DP), jnp.float32),
                  jax.ShapeDtypeStruct((N_EDGES, DP), jnp.float32)),
        mesh=mesh,
        scratch_types=[
            pltpu.VMEM((CHUNKS, C), jnp.int32),
            pltpu.VMEM((2, GE, DP), jnp.float32),
            pltpu.SemaphoreType.DMA,
            pltpu.SemaphoreType.DMA,
            pltpu.SemaphoreType.DMA,
        ],
        compiler_params=pltpu.CompilerParams(use_tc_tiling_on_sc=False),
    )
    return f(h, src2d, tgt2d)


def _sig(x):
    return 1.0 / (1.0 + jnp.exp(-x))


def _in_mlp_body(x_ref, w1_ref, b1_ref, w2_ref, b2_ref, o_ref):
    h1 = jnp.dot(x_ref[...], w1_ref[...], preferred_element_type=jnp.float32)
    h1 = h1 + b1_ref[...]
    h1 = h1 * _sig(h1)
    h2 = jnp.dot(h1, w2_ref[...], preferred_element_type=jnp.float32)
    o_ref[...] = h2 + b2_ref[...]


def _in_mlp(x, W1, b1, W2p, b2p):
    B = 2000
    grid = (N_NODES // B,)
    return pl.pallas_call(
        _in_mlp_body,
        grid=grid,
        in_specs=[
            pl.BlockSpec((B, IN_DIM), lambda i: (i, 0)),
            pl.BlockSpec((IN_DIM, HIDDEN), lambda i: (0, 0)),
            pl.BlockSpec((1, HIDDEN), lambda i: (0, 0)),
            pl.BlockSpec((HIDDEN, DP), lambda i: (0, 0)),
            pl.BlockSpec((1, DP), lambda i: (0, 0)),
        ],
        out_specs=pl.BlockSpec((B, DP), lambda i: (i, 0)),
        out_shape=jax.ShapeDtypeStruct((N_NODES, DP), jnp.float32),
    )(x, W1, b1, W2p, b2p)


NR = N_NODES // NS


def _scatter_body(h_hbm, m_hbm, tgt_hbm, parts_hbm,
                  idxbuf, mbuf, zbuf, acc, msem, ssem):
    c = lax.axis_index("c")
    s = lax.axis_index("s")
    wid = s * NC + c
    ebase = wid * EW

    def zloop(j, carry):
        zbuf[j, :] = jnp.zeros((DP,), jnp.float32)
        return carry

    lax.fori_loop(0, NR, zloop, 0)

    @pl.when(c == 0)
    def _():
        pltpu.sync_copy(h_hbm.at[pl.ds(s * NR, NR)], zbuf)

    pltpu.sync_copy(zbuf, acc.at[pl.ds(s * NR, NR)])
    pltpu.sync_copy(tgt_hbm.at[wid], idxbuf)
    plsc.subcore_barrier()

    ld = pltpu.async_copy(m_hbm.at[pl.ds(ebase, GE)], mbuf.at[0], msem)
    for g in range(NG):
        b = g & 1
        ld.wait()
        if g + 1 < NG:
            ld = pltpu.async_copy(
                m_hbm.at[pl.ds(ebase + (g + 1) * GE, GE)], mbuf.at[b ^ 1],
                msem)

        def fire(j, carry, g=g, b=b):
            pltpu.async_copy(mbuf.at[b, pl.ds(j * C, C)],
                             acc.at[idxbuf.at[g * GC + j]], ssem, add=True)
            return carry

        lax.fori_loop(0, GC, fire, 0)
        pltpu.make_async_copy(mbuf.at[b], acc.at[pl.ds(0, GE)], ssem).wait()

    plsc.subcore_barrier()

    pltpu.sync_copy(acc.at[pl.ds(s * NR, NR)],
                    parts_hbm.at[c, pl.ds(s * NR, NR)])


def _sc_scatter(h, m, tgt3d):
    mesh = plsc.VectorSubcoreMesh(core_axis_name="c", subcore_axis_name="s",
                                  num_cores=NC, num_subcores=NS)
    f = pl.kernel(
        _scatter_body,
        out_type=jax.ShapeDtypeStruct((NC, N_NODES, DP), jnp.float32),
        mesh=mesh,
        scratch_types=[
            pltpu.VMEM((CHUNKS, C), jnp.int32),
            pltpu.VMEM((2, GE, DP), jnp.float32),
            pltpu.VMEM((NR, DP), jnp.float32),
            pltpu.VMEM_SHARED((N_NODES, DP), jnp.float32),
            pltpu.SemaphoreType.DMA,
            pltpu.SemaphoreType.DMA,
        ],
        compiler_params=pltpu.CompilerParams(use_tc_tiling_on_sc=False),
    )
    return f(h, m, tgt3d)


def _add_body(a_ref, b_ref, o_ref):
    o_ref[...] = a_ref[0] + b_ref[0]


def _combine(parts):
    NPK = N_NODES * DP // 128
    return pl.pallas_call(
        _add_body,
        grid=(1,),
        in_specs=[
            pl.BlockSpec((1, NPK, 128), lambda i: (0, 0, 0)),
            pl.BlockSpec((1, NPK, 128), lambda i: (1, 0, 0)),
        ],
        out_specs=pl.BlockSpec((NPK, 128), lambda i: (0, 0)),
        out_shape=jax.ShapeDtypeStruct((NPK, 128), jnp.float32),
    )(parts, parts)


def _edge_body(gs_ref, gt_ref, w1_ref, b1_ref, si_ref, sj_ref, skc_ref,
               wc_ref, w2_ref, b2_ref, o_ref):
    BP = gs_ref.shape[0]
    P_s = gs_ref[...]
    P_t = gt_ref[...]
    hs = jnp.concatenate([P_s[:, DP * j:DP * (j + 1)] for j in range(8)],
                         axis=0)
    ht = jnp.concatenate([P_t[:, DP * j:DP * (j + 1)] for j in range(8)],
                         axis=0)
    hid = jnp.dot(hs, w1_ref[:DP], preferred_element_type=jnp.float32)
    hid = hid + jnp.dot(ht, w1_ref[DP:], preferred_element_type=jnp.float32)
    t1 = jnp.dot(hs, si_ref[...], preferred_element_type=jnp.float32)
    t2 = jnp.dot(ht, sj_ref[...], preferred_element_type=jnp.float32)
    br = jnp.dot(t1 * t2, skc_ref[...], preferred_element_type=jnp.float32)
    hid = hid + jnp.dot(br, wc_ref[...], preferred_element_type=jnp.float32)
    hid = hid + b1_ref[...]
    hid = hid * _sig(hid)
    m = jnp.dot(hid, w2_ref[...], preferred_element_type=jnp.float32)
    m = m + b2_ref[...]
    o_ref[...] = jnp.concatenate([m[BP * j:BP * (j + 1)] for j in range(8)],
                                 axis=1)


def _edge_mlp(gs, gt, W1eff, b1, SI, SJ, SKC, W1c, W2p, b2p):
    B = 3200
    BP = B * DP // 128
    grid = (N_EDGES // B,)
    return pl.pallas_call(
        _edge_body,
        grid=grid,
        in_specs=[
            pl.BlockSpec((BP, 128), lambda i: (i, 0)),
            pl.BlockSpec((BP, 128), lambda i: (i, 0)),
            pl.BlockSpec((2 * DP, HIDDEN), lambda i: (0, 0)),
            pl.BlockSpec((1, HIDDEN), lambda i: (0, 0)),
            pl.BlockSpec((DP, NNZ), lambda i: (0, 0)),
            pl.BlockSpec((DP, NNZ), lambda i: (0, 0)),
            pl.BlockSpec((NNZ, DP), lambda i: (0, 0)),
            pl.BlockSpec((DP, HIDDEN), lambda i: (0, 0)),
            pl.BlockSpec((HIDDEN, DP), lambda i: (0, 0)),
            pl.BlockSpec((1, DP), lambda i: (0, 0)),
        ],
        out_specs=pl.BlockSpec((BP, 128), lambda i: (i, 0)),
        out_shape=jax.ShapeDtypeStruct((N_EDGES * DP // 128, 128),
                                       jnp.float32),
    )(gs, gt, W1eff, b1, SI, SJ, SKC, W1c, W2p, b2p)


def _pool_body(h_ref, k_ref, w1_ref, b1_ref, w2_ref, b2_ref, o_ref):
    hp = h_ref[...]
    sh = jnp.zeros((1, DP), jnp.float32)
    sk = jnp.zeros((1, 1), jnp.float32)
    for j in range(8):
        hj = hp[:, DP * j:DP * (j + 1)]
        hk = jnp.dot(hj, k_ref[...], preferred_element_type=jnp.float32)
        sk = sk + jnp.sum(hk * hj, keepdims=True).reshape(1, 1)
        sh = sh + jnp.sum(hj, axis=0, keepdims=True)
    ph = sh * (1.0 / N_NODES)
    pk = sk * (1.0 / N_NODES)
    p = jnp.concatenate([ph, pk, jnp.zeros((1, 2 * DP - DP - 1), jnp.float32)],
                        axis=1)
    z = jnp.dot(p, w1_ref[...], preferred_element_type=jnp.float32)
    z = z + b1_ref[...]
    z = z * _sig(z)
    o_ref[...] = jnp.dot(z, w2_ref[...],
                         preferred_element_type=jnp.float32) + b2_ref[...]


def _pool(h, Kp, Wo1p, bo1, Wo2, bo2):
    return pl.pallas_call(
        _pool_body,
        in_specs=[
            pl.BlockSpec((N_NODES * DP // 128, 128), lambda: (0, 0)),
            pl.BlockSpec((DP, DP), lambda: (0, 0)),
            pl.BlockSpec((2 * DP, HIDDEN), lambda: (0, 0)),
            pl.BlockSpec((1, HIDDEN), lambda: (0, 0)),
            pl.BlockSpec((HIDDEN, OUT_DIM), lambda: (0, 0)),
            pl.BlockSpec((1, OUT_DIM), lambda: (0, 0)),
        ],
        out_specs=pl.BlockSpec((1, OUT_DIM), lambda: (0, 0)),
        out_shape=jax.ShapeDtypeStruct((1, OUT_DIM), jnp.float32),
    )(h, Kp, Wo1p, bo1, Wo2, bo2)


def kernel(x, edge_index, W_in1, b_in1, W_in2, b_in2, l0_W1, l0_b1, l0_W2,
           l0_b2, l1_W1, l1_b1, l1_W2, l1_b2, Kmat, out_W1, out_b1, out_W2,
           out_b2, bI, bJ, bK, bC):
    src = edge_index[0]
    tgt = edge_index[1]

    pad = lambda a, r: jnp.zeros((r, a.shape[1]), a.dtype).at[:a.shape[0]].set(a)
    padc = lambda a, c: jnp.zeros((a.shape[0], c), a.dtype).at[:, :a.shape[1]].set(a)

    W_in2p = padc(W_in2, DP)
    b_in2p = padc(b_in2[None, :], DP)

    SI = jnp.zeros((DP, NNZ), jnp.float32).at[bI, jnp.arange(NNZ)].set(1.0)
    SJ = jnp.zeros((DP, NNZ), jnp.float32).at[bJ, jnp.arange(NNZ)].set(1.0)
    SKC = jnp.zeros((NNZ, DP), jnp.float32).at[jnp.arange(NNZ), bK].set(bC)

    layers = []
    for (W1, b1, W2, b2) in ((l0_W1, l0_b1, l0_W2, l0_b2),
                             (l1_W1, l1_b1, l1_W2, l1_b2)):
        W1eff = jnp.concatenate([pad(W1[:D_A], DP), pad(W1[D_A:2 * D_A], DP)],
                                axis=0)
        W1c = pad(W1[2 * D_A:], DP)
        W2p = padc(W2, DP)
        b2p = padc(b2[None, :], DP)
        layers.append((W1eff, b1[None, :], W1c, W2p, b2p))

    Kp = pad(padc(Kmat, DP), DP)
    Wo1p = pad(out_W1, 2 * DP).at[D_A + 1:].set(0.0)
    Wo1p = jnp.zeros((2 * DP, HIDDEN), jnp.float32)
    Wo1p = Wo1p.at[:D_A].set(out_W1[:D_A])
    Wo1p = Wo1p.at[DP].set(out_W1[D_A])

    src2d = src.reshape(NW, CHUNKS, C)
    tgt2d = tgt.reshape(NW, CHUNKS, C)
    hp = _in_mlp(x, W_in1, b_in1[None, :], W_in2p, b_in2p)

    EP = N_EDGES * DP // 128
    NPK = N_NODES * DP // 128
    for (W1eff, b1r, W1c, W2p, b2p) in layers:
        hu = jnp.reshape(hp, (N_NODES, DP))
        gs, gt = _sc_gather(hu, src2d, tgt2d)
        m_p = _edge_mlp(jnp.reshape(gs, (EP, 128)),
                        jnp.reshape(gt, (EP, 128)),
                        W1eff, b1r, SI, SJ, SKC, W1c, W2p, b2p)
        parts = _sc_scatter(hu, jnp.reshape(m_p, (N_EDGES, DP)), tgt2d)
        hp = _combine(jnp.reshape(parts, (NC, NPK, 128)))

    return _pool(hp, Kp, Wo1p, out_b1[None, :], out_W2, out_b2[None, :])


# speedup vs baseline: 18.2882x; 1.1119x over previous
import functools
import jax
import jax.numpy as jnp
import numpy as np
from jax import lax
from jax.experimental import pallas as pl
from jax.experimental.pallas import tpu as pltpu
from jax.experimental.pallas import tpu_sc as plsc

N_NODES = 10000
N_EDGES = 320000
IN_DIM = 128
HIDDEN = 128
OUT_DIM = 32
D_A = 14
DP = 16
NNZ = 64

NC = 2
NS = 16
NW = NC * NS
EW = N_EDGES // NW
C = 80
CHUNKS = EW // C
GC = 25
GE = GC * C
NG = CHUNKS // GC


def _gather_body(h_hbm, src_hbm, tgt_hbm, gs_hbm, gt_hbm,
                 idxbuf, rows, gsem, osem0, osem1):
    c = lax.axis_index("c")
    s = lax.axis_index("s")
    wid = s * NC + c
    ebase = wid * EW
    for idx_hbm, out_hbm in ((src_hbm, gs_hbm), (tgt_hbm, gt_hbm)):
        pltpu.sync_copy(idx_hbm.at[wid], idxbuf)
        osems = (osem0, osem1)
        descs = [None] * NG
        for g in range(NG):
            b = g & 1
            if g >= 2:
                descs[g - 2].wait()

            def fire(j, carry, g=g, b=b):
                pltpu.async_copy(h_hbm.at[idxbuf.at[g * GC + j]],
                                 rows.at[b, pl.ds(j * C, C)], gsem)
                return carry

            lax.fori_loop(0, GC, fire, 0)
            pltpu.make_async_copy(out_hbm.at[pl.ds(ebase + g * GE, GE)],
                                  rows.at[b], gsem).wait()
            descs[g] = pltpu.async_copy(
                rows.at[b], out_hbm.at[pl.ds(ebase + g * GE, GE)], osems[b])
        descs[NG - 2].wait()
        descs[NG - 1].wait()


def _sc_gather(h, src2d, tgt2d):
    mesh = plsc.VectorSubcoreMesh(core_axis_name="c", subcore_axis_name="s",
                                  num_cores=NC, num_subcores=NS)
    f = pl.kernel(
        _gather_body,
        out_type=(jax.ShapeDtypeStruct((N_EDGES, DP), jnp.float32),
                  jax.ShapeDtypeStruct((N_EDGES, DP), jnp.float32)),
        mesh=mesh,
        scratch_types=[
            pltpu.VMEM((CHUNKS, C), jnp.int32),
            pltpu.VMEM((2, GE, DP), jnp.float32),
            pltpu.SemaphoreType.DMA,
            pltpu.SemaphoreType.DMA,
            pltpu.SemaphoreType.DMA,
        ],
        compiler_params=pltpu.CompilerParams(use_tc_tiling_on_sc=False),
    )
    return f(h, src2d, tgt2d)


def _sig(x):
    return 1.0 / (1.0 + jnp.exp(-x))


def _in_mlp_body(x_ref, w1_ref, b1_ref, w2_ref, b2_ref, o_ref):
    h1 = jnp.dot(x_ref[...], w1_ref[...], preferred_element_type=jnp.float32)
    h1 = h1 + b1_ref[...]
    h1 = h1 * _sig(h1)
    h2 = jnp.dot(h1, w2_ref[...], preferred_element_type=jnp.float32)
    o_ref[...] = h2 + b2_ref[...]


def _in_mlp(x, W1, b1, W2p, b2p):
    B = 2000
    grid = (N_NODES // B,)
    return pl.pallas_call(
        _in_mlp_body,
        grid=grid,
        in_specs=[
            pl.BlockSpec((B, IN_DIM), lambda i: (i, 0)),
            pl.BlockSpec((IN_DIM, HIDDEN), lambda i: (0, 0)),
            pl.BlockSpec((1, HIDDEN), lambda i: (0, 0)),
            pl.BlockSpec((HIDDEN, DP), lambda i: (0, 0)),
            pl.BlockSpec((1, DP), lambda i: (0, 0)),
        ],
        out_specs=pl.BlockSpec((B, DP), lambda i: (i, 0)),
        out_shape=jax.ShapeDtypeStruct((N_NODES, DP), jnp.float32),
    )(x, W1, b1, W2p, b2p)


NR = N_NODES // NS


def _scatter_body(h_hbm, m_hbm, tgt_hbm, parts_hbm,
                  idxbuf, mbuf, zbuf, acc, msem, ssem):
    c = lax.axis_index("c")
    s = lax.axis_index("s")
    wid = s * NC + c
    ebase = wid * EW

    def zloop(j, carry):
        zbuf[j, :] = jnp.zeros((DP,), jnp.float32)
        return carry

    lax.fori_loop(0, NR, zloop, 0)

    @pl.when(c == 0)
    def _():
        pltpu.sync_copy(h_hbm.at[pl.ds(s * NR, NR)], zbuf)

    pltpu.sync_copy(zbuf, acc.at[pl.ds(s * NR, NR)])
    pltpu.sync_copy(tgt_hbm.at[wid], idxbuf)
    plsc.subcore_barrier()

    ld = pltpu.async_copy(m_hbm.at[pl.ds(ebase, GE)], mbuf.at[0], msem)
    for g in range(NG):
        b = g & 1
        ld.wait()
        if g + 1 < NG:
            ld = pltpu.async_copy(
                m_hbm.at[pl.ds(ebase + (g + 1) * GE, GE)], mbuf.at[b ^ 1],
                msem)

        def fire(j, carry, g=g, b=b):
            pltpu.async_copy(mbuf.at[b, pl.ds(j * C, C)],
                             acc.at[idxbuf.at[g * GC + j]], ssem, add=True)
            return carry

        lax.fori_loop(0, GC, fire, 0)
        pltpu.make_async_copy(mbuf.at[b], acc.at[pl.ds(0, GE)], ssem).wait()

    plsc.subcore_barrier()

    pltpu.sync_copy(acc.at[pl.ds(s * NR, NR)],
                    parts_hbm.at[c, pl.ds(s * NR, NR)])


def _sc_scatter(h, m, tgt3d):
    mesh = plsc.VectorSubcoreMesh(core_axis_name="c", subcore_axis_name="s",
                                  num_cores=NC, num_subcores=NS)
    f = pl.kernel(
        _scatter_body,
        out_type=jax.ShapeDtypeStruct((NC, N_NODES, DP), jnp.float32),
        mesh=mesh,
        scratch_types=[
            pltpu.VMEM((CHUNKS, C), jnp.int32),
            pltpu.VMEM((2, GE, DP), jnp.float32),
            pltpu.VMEM((NR, DP), jnp.float32),
            pltpu.VMEM_SHARED((N_NODES, DP), jnp.float32),
            pltpu.SemaphoreType.DMA,
            pltpu.SemaphoreType.DMA,
        ],
        compiler_params=pltpu.CompilerParams(use_tc_tiling_on_sc=False),
    )
    return f(h, m, tgt3d)


def _add_body(a_ref, b_ref, o_ref):
    o_ref[...] = a_ref[0] + b_ref[0]


def _combine(parts):
    NPK = N_NODES * DP // 128
    return pl.pallas_call(
        _add_body,
        grid=(1,),
        in_specs=[
            pl.BlockSpec((1, NPK, 128), lambda i: (0, 0, 0)),
            pl.BlockSpec((1, NPK, 128), lambda i: (1, 0, 0)),
        ],
        out_specs=pl.BlockSpec((NPK, 128), lambda i: (0, 0)),
        out_shape=jax.ShapeDtypeStruct((NPK, 128), jnp.float32),
    )(parts, parts)


def _edge_body(gs_ref, gt_ref, wbig_ref, b1_ref, wbr_ref, w2_ref, b2_ref,
               o_ref):
    BP = gs_ref.shape[0]
    B = BP * 8
    P_s = gs_ref[...]
    P_t = gt_ref[...]
    hsht = jnp.concatenate(
        [jnp.concatenate([P_s[:, DP * j:DP * (j + 1)],
                          P_t[:, DP * j:DP * (j + 1)]], axis=1)
         for j in range(8)], axis=0)
    y = jnp.dot(hsht, wbig_ref[...], preferred_element_type=jnp.float32)
    hid = y[:, :HIDDEN]
    t1 = y[:, HIDDEN:HIDDEN + NNZ]
    t2 = y[:, HIDDEN + NNZ:]
    hid = hid + jnp.dot(t1 * t2, wbr_ref[...],
                        preferred_element_type=jnp.float32)
    hid = hid + b1_ref[...]
    hid = hid * _sig(hid)
    m = jnp.dot(hid, w2_ref[...], preferred_element_type=jnp.float32)
    m = m + b2_ref[...]
    o_ref[...] = jnp.concatenate([m[BP * j:BP * (j + 1)] for j in range(8)],
                                 axis=1)


def _edge_mlp(gs, gt, Wbig, b1, Wbr, W2p, b2p):
    B = 3200
    BP = B * DP // 128
    grid = (N_EDGES // B,)
    return pl.pallas_call(
        _edge_body,
        grid=grid,
        in_specs=[
            pl.BlockSpec((BP, 128), lambda i: (i, 0)),
            pl.BlockSpec((BP, 128), lambda i: (i, 0)),
            pl.BlockSpec((2 * DP, HIDDEN + 2 * NNZ), lambda i: (0, 0)),
            pl.BlockSpec((1, HIDDEN), lambda i: (0, 0)),
            pl.BlockSpec((NNZ, HIDDEN), lambda i: (0, 0)),
            pl.BlockSpec((HIDDEN, DP), lambda i: (0, 0)),
            pl.BlockSpec((1, DP), lambda i: (0, 0)),
        ],
        out_specs=pl.BlockSpec((BP, 128), lambda i: (i, 0)),
        out_shape=jax.ShapeDtypeStruct((N_EDGES * DP // 128, 128),
                                       jnp.float32),
    )(gs, gt, Wbig, b1, Wbr, W2p, b2p)


def _pool_body(h_ref, k_ref, w1_ref, b1_ref, w2_ref, b2_ref, o_ref):
    hp = h_ref[...]
    sh = jnp.zeros((1, DP), jnp.float32)
    sk = jnp.zeros((1, 1), jnp.float32)
    for j in range(8):
        hj = hp[:, DP * j:DP * (j + 1)]
        hk = jnp.dot(hj, k_ref[...], preferred_element_type=jnp.float32)
        sk = sk + jnp.sum(hk * hj, keepdims=True).reshape(1, 1)
        sh = sh + jnp.sum(hj, axis=0, keepdims=True)
    ph = sh * (1.0 / N_NODES)
    pk = sk * (1.0 / N_NODES)
    p = jnp.concatenate([ph, pk, jnp.zeros((1, 2 * DP - DP - 1), jnp.float32)],
                        axis=1)
    z = jnp.dot(p, w1_ref[...], preferred_element_type=jnp.float32)
    z = z + b1_ref[...]
    z = z * _sig(z)
    o_ref[...] = jnp.dot(z, w2_ref[...],
                         preferred_element_type=jnp.float32) + b2_ref[...]


def _pool(h, Kp, Wo1p, bo1, Wo2, bo2):
    return pl.pallas_call(
        _pool_body,
        in_specs=[
            pl.BlockSpec((N_NODES * DP // 128, 128), lambda: (0, 0)),
            pl.BlockSpec((DP, DP), lambda: (0, 0)),
            pl.BlockSpec((2 * DP, HIDDEN), lambda: (0, 0)),
            pl.BlockSpec((1, HIDDEN), lambda: (0, 0)),
            pl.BlockSpec((HIDDEN, OUT_DIM), lambda: (0, 0)),
            pl.BlockSpec((1, OUT_DIM), lambda: (0, 0)),
        ],
        out_specs=pl.BlockSpec((1, OUT_DIM), lambda: (0, 0)),
        out_shape=jax.ShapeDtypeStruct((1, OUT_DIM), jnp.float32),
    )(h, Kp, Wo1p, bo1, Wo2, bo2)


def kernel(x, edge_index, W_in1, b_in1, W_in2, b_in2, l0_W1, l0_b1, l0_W2,
           l0_b2, l1_W1, l1_b1, l1_W2, l1_b2, Kmat, out_W1, out_b1, out_W2,
           out_b2, bI, bJ, bK, bC):
    src = edge_index[0]
    tgt = edge_index[1]

    def padr(a, r):
        return jnp.concatenate(
            [a, jnp.zeros((r - a.shape[0], a.shape[1]), a.dtype)], axis=0)

    def padc(a, c):
        return jnp.concatenate(
            [a, jnp.zeros((a.shape[0], c - a.shape[1]), a.dtype)], axis=1)

    W_in2p = padc(W_in2, DP)
    b_in2p = padc(b_in2[None, :], DP)

    zDN = jnp.zeros((DP, NNZ), jnp.float32)
    SI = (lax.broadcasted_iota(jnp.int32, (DP, NNZ), 0) ==
          bI[None, :]).astype(jnp.float32)
    SJ = (lax.broadcasted_iota(jnp.int32, (DP, NNZ), 0) ==
          bJ[None, :]).astype(jnp.float32)

    layers = []
    for (W1, b1, W2, b2) in ((l0_W1, l0_b1, l0_W2, l0_b2),
                             (l1_W1, l1_b1, l1_W2, l1_b2)):
        W1eff = jnp.concatenate([padr(W1[:D_A], DP),
                                 padr(W1[D_A:2 * D_A], DP)],
                                axis=0)
        Wbig = jnp.concatenate(
            [W1eff,
             jnp.concatenate([SI, zDN], axis=0),
             jnp.concatenate([zDN, SJ], axis=0)], axis=1)
        Wbr = bC[:, None] * jnp.take(W1[2 * D_A:], bK, axis=0)
        W2p = padc(W2, DP)
        b2p = padc(b2[None, :], DP)
        layers.append((Wbig, b1[None, :], Wbr, W2p, b2p))

    Kp = padr(padc(Kmat, DP), DP)
    Wo1p = jnp.concatenate(
        [out_W1[:D_A], jnp.zeros((DP - D_A, HIDDEN), jnp.float32),
         out_W1[D_A:D_A + 1], jnp.zeros((2 * DP - DP - 1, HIDDEN),
                                        jnp.float32)], axis=0)

    src2d = src.reshape(NW, CHUNKS, C)
    tgt2d = tgt.reshape(NW, CHUNKS, C)
    hp = _in_mlp(x, W_in1, b_in1[None, :], W_in2p, b_in2p)

    EP = N_EDGES * DP // 128
    NPK = N_NODES * DP // 128
    for (Wbig, b1r, Wbr, W2p, b2p) in layers:
        hu = jnp.reshape(hp, (N_NODES, DP))
        gs, gt = _sc_gather(hu, src2d, tgt2d)
        m_p = _edge_mlp(jnp.reshape(gs, (EP, 128)),
                        jnp.reshape(gt, (EP, 128)),
                        Wbig, b1r, Wbr, W2p, b2p)
        parts = _sc_scatter(hu, jnp.reshape(m_p, (N_EDGES, DP)), tgt2d)
        hp = _combine(jnp.reshape(parts, (NC, NPK, 128)))

    return _pool(hp, Kp, Wo1p, out_b1[None, :], out_W2, out_b2[None, :])


# speedup vs baseline: 20.0386x; 1.0957x over previous
import functools
import jax
import jax.numpy as jnp
import numpy as np
from jax import lax
from jax.experimental import pallas as pl
from jax.experimental.pallas import tpu as pltpu
from jax.experimental.pallas import tpu_sc as plsc

N_NODES = 10000
N_EDGES = 320000
IN_DIM = 128
HIDDEN = 128
OUT_DIM = 32
D_A = 14
DP = 16
NNZ = 64

NC = 2
NS = 16
NW = NC * NS
EW = N_EDGES // NW
C = 80
CHUNKS = EW // C
GC = 25
GE = GC * C
NG = CHUNKS // GC


def _gather_body(h_hbm, src_hbm, tgt_hbm, gs_hbm, gt_hbm,
                 idxbuf, rows, gsem, osem0, osem1):
    c = lax.axis_index("c")
    s = lax.axis_index("s")
    wid = s * NC + c
    ebase = wid * EW
    for idx_hbm, out_hbm in ((src_hbm, gs_hbm), (tgt_hbm, gt_hbm)):
        pltpu.sync_copy(idx_hbm.at[wid], idxbuf)
        osems = (osem0, osem1)
        descs = [None] * NG
        for g in range(NG):
            b = g & 1
            if g >= 2:
                descs[g - 2].wait()

            def fire(j, carry, g=g, b=b):
                pltpu.async_copy(h_hbm.at[idxbuf.at[g * GC + j]],
                                 rows.at[b, pl.ds(j * C, C)], gsem)
                return carry

            lax.fori_loop(0, GC, fire, 0)
            pltpu.make_async_copy(out_hbm.at[pl.ds(ebase + g * GE, GE)],
                                  rows.at[b], gsem).wait()
            descs[g] = pltpu.async_copy(
                rows.at[b], out_hbm.at[pl.ds(ebase + g * GE, GE)], osems[b])
        descs[NG - 2].wait()
        descs[NG - 1].wait()


def _sc_gather(h, src2d, tgt2d):
    mesh = plsc.VectorSubcoreMesh(core_axis_name="c", subcore_axis_name="s",
                                  num_cores=NC, num_subcores=NS)
    f = pl.kernel(
        _gather_body,
        out_type=(jax.ShapeDtypeStruct((N_EDGES, DP), jnp.float32),
                  jax.ShapeDtypeStruct((N_EDGES, DP), jnp.float32)),
        mesh=mesh,
        scratch_types=[
            pltpu.VMEM((CHUNKS, C), jnp.int32),
            pltpu.VMEM((2, GE, DP), jnp.float32),
            pltpu.SemaphoreType.DMA,
            pltpu.SemaphoreType.DMA,
            pltpu.SemaphoreType.DMA,
        ],
        compiler_params=pltpu.CompilerParams(use_tc_tiling_on_sc=False),
    )
    return f(h, src2d, tgt2d)


def _sig(x):
    return 1.0 / (1.0 + jnp.exp(-x))


def _in_mlp_body(x_ref, w1_ref, b1_ref, w2_ref, b2_ref, o_ref):
    h1 = jnp.dot(x_ref[...], w1_ref[...], preferred_element_type=jnp.float32)
    h1 = h1 + b1_ref[...]
    h1 = h1 * _sig(h1)
    h2 = jnp.dot(h1, w2_ref[...], preferred_element_type=jnp.float32)
    o_ref[...] = h2 + b2_ref[...]


def _in_mlp(x, W1, b1, W2p, b2p):
    B = 2000
    grid = (N_NODES // B,)
    return pl.pallas_call(
        _in_mlp_body,
        grid=grid,
        in_specs=[
            pl.BlockSpec((B, IN_DIM), lambda i: (i, 0)),
            pl.BlockSpec((IN_DIM, HIDDEN), lambda i: (0, 0)),
            pl.BlockSpec((1, HIDDEN), lambda i: (0, 0)),
            pl.BlockSpec((HIDDEN, DP), lambda i: (0, 0)),
            pl.BlockSpec((1, DP), lambda i: (0, 0)),
        ],
        out_specs=pl.BlockSpec((B, DP), lambda i: (i, 0)),
        out_shape=jax.ShapeDtypeStruct((N_NODES, DP), jnp.float32),
    )(x, W1, b1, W2p, b2p)


NR = N_NODES // NS


def _scatter_body(h_hbm, m_hbm, tgt_hbm, parts_hbm,
                  idxbuf, mbuf, zbuf, acc, msem, ssem):
    c = lax.axis_index("c")
    s = lax.axis_index("s")
    wid = s * NC + c
    ebase = wid * EW

    def zloop(j, carry):
        zbuf[j, :] = jnp.zeros((DP,), jnp.float32)
        return carry

    lax.fori_loop(0, NR, zloop, 0)

    @pl.when(c == 0)
    def _():
        pltpu.sync_copy(h_hbm.at[pl.ds(s * NR, NR)], zbuf)

    pltpu.sync_copy(zbuf, acc.at[pl.ds(s * NR, NR)])
    pltpu.sync_copy(tgt_hbm.at[wid], idxbuf)
    plsc.subcore_barrier()

    ld = pltpu.async_copy(m_hbm.at[pl.ds(ebase, GE)], mbuf.at[0], msem)
    for g in range(NG):
        b = g & 1
        ld.wait()
        if g + 1 < NG:
            ld = pltpu.async_copy(
                m_hbm.at[pl.ds(ebase + (g + 1) * GE, GE)], mbuf.at[b ^ 1],
                msem)

        def fire(j, carry, g=g, b=b):
            pltpu.async_copy(mbuf.at[b, pl.ds(j * C, C)],
                             acc.at[idxbuf.at[g * GC + j]], ssem, add=True)
            return carry

        lax.fori_loop(0, GC, fire, 0)
        pltpu.make_async_copy(mbuf.at[b], acc.at[pl.ds(0, GE)], ssem).wait()

    plsc.subcore_barrier()

    pltpu.sync_copy(acc.at[pl.ds(s * NR, NR)],
                    parts_hbm.at[c, pl.ds(s * NR, NR)])


def _sc_scatter(h, m, tgt3d):
    mesh = plsc.VectorSubcoreMesh(core_axis_name="c", subcore_axis_name="s",
                                  num_cores=NC, num_subcores=NS)
    f = pl.kernel(
        _scatter_body,
        out_type=jax.ShapeDtypeStruct((NC, N_NODES, DP), jnp.float32),
        mesh=mesh,
        scratch_types=[
            pltpu.VMEM((CHUNKS, C), jnp.int32),
            pltpu.VMEM((2, GE, DP), jnp.float32),
            pltpu.VMEM((NR, DP), jnp.float32),
            pltpu.VMEM_SHARED((N_NODES, DP), jnp.float32),
            pltpu.SemaphoreType.DMA,
            pltpu.SemaphoreType.DMA,
        ],
        compiler_params=pltpu.CompilerParams(use_tc_tiling_on_sc=False),
    )
    return f(h, m, tgt3d)


def _add_body(a_ref, b_ref, o_ref):
    o_ref[...] = a_ref[0] + b_ref[0]


def _combine(parts):
    NPK = N_NODES * DP // 128
    return pl.pallas_call(
        _add_body,
        grid=(1,),
        in_specs=[
            pl.BlockSpec((1, NPK, 128), lambda i: (0, 0, 0)),
            pl.BlockSpec((1, NPK, 128), lambda i: (1, 0, 0)),
        ],
        out_specs=pl.BlockSpec((NPK, 128), lambda i: (0, 0)),
        out_shape=jax.ShapeDtypeStruct((NPK, 128), jnp.float32),
    )(parts, parts)


HID8 = 8 * HIDDEN
TRM8 = 8 * NNZ


def _edge_body(gs_ref, gt_ref, wbig_ref, b1_ref, wbr_ref, w2_ref, b2_ref,
               o_ref):
    x = jnp.concatenate([gs_ref[...], gt_ref[...]], axis=1)
    y = jnp.dot(x, wbig_ref[...], preferred_element_type=jnp.float32)
    hid = y[:, :HID8]
    tt = y[:, HID8:HID8 + TRM8] * y[:, HID8 + TRM8:]
    hid = hid + jnp.dot(tt, wbr_ref[...], preferred_element_type=jnp.float32)
    hid = hid + b1_ref[...]
    hid = hid * _sig(hid)
    m = jnp.dot(hid, w2_ref[...], preferred_element_type=jnp.float32)
    o_ref[...] = m + b2_ref[...]


def _edge_mlp(gs, gt, Wbig, b1, Wbr, W2p, b2p):
    B = 3200
    BP = B * DP // 128
    grid = (N_EDGES // B,)
    return pl.pallas_call(
        _edge_body,
        grid=grid,
        in_specs=[
            pl.BlockSpec((BP, 128), lambda i: (i, 0)),
            pl.BlockSpec((BP, 128), lambda i: (i, 0)),
            pl.BlockSpec((2 * 128, HID8 + 2 * TRM8), lambda i: (0, 0)),
            pl.BlockSpec((1, HID8), lambda i: (0, 0)),
            pl.BlockSpec((TRM8, HID8), lambda i: (0, 0)),
            pl.BlockSpec((HID8, 128), lambda i: (0, 0)),
            pl.BlockSpec((1, 128), lambda i: (0, 0)),
        ],
        out_specs=pl.BlockSpec((BP, 128), lambda i: (i, 0)),
        out_shape=jax.ShapeDtypeStruct((N_EDGES * DP // 128, 128),
                                       jnp.float32),
    )(gs, gt, Wbig, b1, Wbr, W2p, b2p)


def _pool_body(h_ref, k_ref, w1_ref, b1_ref, w2_ref, b2_ref, o_ref):
    hp = h_ref[...]
    sh = jnp.zeros((1, DP), jnp.float32)
    sk = jnp.zeros((1, 1), jnp.float32)
    for j in range(8):
        hj = hp[:, DP * j:DP * (j + 1)]
        hk = jnp.dot(hj, k_ref[...], preferred_element_type=jnp.float32)
        sk = sk + jnp.sum(hk * hj, keepdims=True).reshape(1, 1)
        sh = sh + jnp.sum(hj, axis=0, keepdims=True)
    ph = sh * (1.0 / N_NODES)
    pk = sk * (1.0 / N_NODES)
    p = jnp.concatenate([ph, pk, jnp.zeros((1, 2 * DP - DP - 1), jnp.float32)],
                        axis=1)
    z = jnp.dot(p, w1_ref[...], preferred_element_type=jnp.float32)
    z = z + b1_ref[...]
    z = z * _sig(z)
    o_ref[...] = jnp.dot(z, w2_ref[...],
                         preferred_element_type=jnp.float32) + b2_ref[...]


def _pool(h, Kp, Wo1p, bo1, Wo2, bo2):
    return pl.pallas_call(
        _pool_body,
        in_specs=[
            pl.BlockSpec((N_NODES * DP // 128, 128), lambda: (0, 0)),
            pl.BlockSpec((DP, DP), lambda: (0, 0)),
            pl.BlockSpec((2 * DP, HIDDEN), lambda: (0, 0)),
            pl.BlockSpec((1, HIDDEN), lambda: (0, 0)),
            pl.BlockSpec((HIDDEN, OUT_DIM), lambda: (0, 0)),
            pl.BlockSpec((1, OUT_DIM), lambda: (0, 0)),
        ],
        out_specs=pl.BlockSpec((1, OUT_DIM), lambda: (0, 0)),
        out_shape=jax.ShapeDtypeStruct((1, OUT_DIM), jnp.float32),
    )(h, Kp, Wo1p, bo1, Wo2, bo2)


def kernel(x, edge_index, W_in1, b_in1, W_in2, b_in2, l0_W1, l0_b1, l0_W2,
           l0_b2, l1_W1, l1_b1, l1_W2, l1_b2, Kmat, out_W1, out_b1, out_W2,
           out_b2, bI, bJ, bK, bC):
    src = edge_index[0]
    tgt = edge_index[1]

    def padr(a, r):
        return jnp.concatenate(
            [a, jnp.zeros((r - a.shape[0], a.shape[1]), a.dtype)], axis=0)

    def padc(a, c):
        return jnp.concatenate(
            [a, jnp.zeros((a.shape[0], c - a.shape[1]), a.dtype)], axis=1)

    W_in2p = padc(W_in2, DP)
    b_in2p = padc(b_in2[None, :], DP)

    zDN = jnp.zeros((DP, NNZ), jnp.float32)
    SI = (lax.broadcasted_iota(jnp.int32, (DP, NNZ), 0) ==
          bI[None, :]).astype(jnp.float32)
    SJ = (lax.broadcasted_iota(jnp.int32, (DP, NNZ), 0) ==
          bJ[None, :]).astype(jnp.float32)

    I8 = jnp.eye(8, dtype=jnp.float32)
    zBD = jnp.zeros((128, TRM8), jnp.float32)
    layers = []
    for (W1, b1, W2, b2) in ((l0_W1, l0_b1, l0_W2, l0_b2),
                             (l1_W1, l1_b1, l1_W2, l1_b2)):
        top = jnp.concatenate([jnp.kron(I8, padr(W1[:D_A], DP)),
                               jnp.kron(I8, SI), zBD], axis=1)
        bot = jnp.concatenate([jnp.kron(I8, padr(W1[D_A:2 * D_A], DP)),
                               zBD, jnp.kron(I8, SJ)], axis=1)
        Wbig = jnp.concatenate([top, bot], axis=0)
        Wbr = bC[:, None] * jnp.take(W1[2 * D_A:], bK, axis=0)
        WbrBD = jnp.kron(I8, Wbr)
        W2BD = jnp.kron(I8, padc(W2, DP))
        b1t = jnp.tile(b1[None, :], (1, 8))
        b2t = jnp.tile(padc(b2[None, :], DP), (1, 8))
        layers.append((Wbig, b1t, WbrBD, W2BD, b2t))

    Kp = padr(padc(Kmat, DP), DP)
    Wo1p = jnp.concatenate(
        [out_W1[:D_A], jnp.zeros((DP - D_A, HIDDEN), jnp.float32),
         out_W1[D_A:D_A + 1], jnp.zeros((2 * DP - DP - 1, HIDDEN),
                                        jnp.float32)], axis=0)

    src2d = src.reshape(NW, CHUNKS, C)
    tgt2d = tgt.reshape(NW, CHUNKS, C)
    hp = _in_mlp(x, W_in1, b_in1[None, :], W_in2p, b_in2p)

    EP = N_EDGES * DP // 128
    NPK = N_NODES * DP // 128
    for (Wbig, b1t, WbrBD, W2BD, b2t) in layers:
        hu = jnp.reshape(hp, (N_NODES, DP))
        gs, gt = _sc_gather(hu, src2d, tgt2d)
        m_p = _edge_mlp(jnp.reshape(gs, (EP, 128)),
                        jnp.reshape(gt, (EP, 128)),
                        Wbig, b1t, WbrBD, W2BD, b2t)
        parts = _sc_scatter(hu, jnp.reshape(m_p, (N_EDGES, DP)), tgt2d)
        hp = _combine(jnp.reshape(parts, (NC, NPK, 128)))

    return _pool(hp, Kp, Wo1p, out_b1[None, :], out_W2, out_b2[None, :])
